# baseline (speedup 1.0000x reference)
import jax
import jax.numpy as jnp
from jax.experimental import pallas as pl
from jax.experimental.pallas import tpu as pltpu

_LIMIT = float(jnp.pi) / 2.0
_LIMIT_SQ = _LIMIT * _LIMIT
_WEIGHT = 0.01

_N = 16384
_D = 72
_BANDS = (1, 3)
_BAND_ROWS = ((1, 2), (0,))


def _tc_body(a_ref, o_ref, acc_ref):
    i = pl.program_id(0)

    @pl.when(i == 0)
    def _():
        acc_ref[0] = 0.0

    v = a_ref[...]
    r = jax.lax.broadcasted_iota(jnp.int32, v.shape, 0)
    is0 = i == 0
    rowmask = (((r == 1) | (r == 2)) & is0) | ((r == 0) & jnp.logical_not(is0))
    sq = v * v
    keep = rowmask & (sq > _LIMIT_SQ)
    acc_ref[0] += jnp.sum(jnp.where(keep, sq, 0.0))

    @pl.when(i == 1)
    def _():
        o_ref[0] = acc_ref[0] * _WEIGHT


@jax.jit
def kernel(pose):
    xt = pose.T
    out = pl.pallas_call(
        _tc_body,
        grid=(2,),
        in_specs=[pl.BlockSpec((8, _N), lambda i: (1 + 2 * i, 0))],
        out_specs=pl.BlockSpec(memory_space=pltpu.SMEM),
        out_shape=jax.ShapeDtypeStruct((1,), jnp.float32),
        scratch_shapes=[pltpu.SMEM((1,), jnp.float32)],
        compiler_params=pltpu.CompilerParams(
            dimension_semantics=("arbitrary",),
        ),
    )(xt)
    return out[0]


# speedup vs baseline: 1.2604x; 1.2604x over previous
import jax
import jax.numpy as jnp
from jax.experimental import pallas as pl
from jax.experimental.pallas import tpu as pltpu

_LIMIT = float(jnp.pi) / 2.0
_LIMIT_SQ = _LIMIT * _LIMIT
_WEIGHT = 0.01

_N = 16384
_D = 72


def _tc_body(x_hbm, o_ref, buf, sem_a, sem_b):
    cp_a = pltpu.make_async_copy(
        x_hbm.at[pl.ds(9, 2), :], buf.at[pl.ds(0, 2), :], sem_a
    )
    cp_b = pltpu.make_async_copy(
        x_hbm.at[pl.ds(24, 1), :], buf.at[pl.ds(2, 1), :], sem_b
    )
    cp_a.start()
    cp_b.start()
    cp_a.wait()
    cp_b.wait()

    v = buf[...]
    r = jax.lax.broadcasted_iota(jnp.int32, v.shape, 0)
    sq = v * v
    keep = (r < 3) & (sq > _LIMIT_SQ)
    o_ref[0] = jnp.sum(jnp.where(keep, sq, 0.0)) * _WEIGHT


@jax.jit
def kernel(pose):
    xt = pose.T
    out = pl.pallas_call(
        _tc_body,
        grid=(1,),
        in_specs=[pl.BlockSpec(memory_space=pl.ANY)],
        out_specs=pl.BlockSpec(memory_space=pltpu.SMEM),
        out_shape=jax.ShapeDtypeStruct((1,), jnp.float32),
        scratch_shapes=[
            pltpu.VMEM((8, _N), jnp.float32),
            pltpu.SemaphoreType.DMA,
            pltpu.SemaphoreType.DMA,
        ],
    )(xt)
    return out[0]


# speedup vs baseline: 1.2948x; 1.0273x over previous
import jax
import jax.numpy as jnp
from jax.experimental import pallas as pl
from jax.experimental.pallas import tpu as pltpu

_LIMIT = float(jnp.pi) / 2.0
_LIMIT_SQ = _LIMIT * _LIMIT
_WEIGHT = 0.01

_N = 16384
_D = 72


_H = _N // 2


def _tc_body(x_hbm, o_ref, buf, sem):
    cps = []
    for k, row in enumerate((9, 10, 24)):
        for h in range(2):
            cps.append(
                pltpu.make_async_copy(
                    x_hbm.at[pl.ds(row, 1), pl.ds(h * _H, _H)],
                    buf.at[pl.ds(2 * k + h, 1), :],
                    sem,
                )
            )
    for cp in cps:
        cp.start()
    for cp in cps:
        cp.wait()

    v = buf[...]
    r = jax.lax.broadcasted_iota(jnp.int32, v.shape, 0)
    sq = v * v
    keep = (r < 6) & (sq > _LIMIT_SQ)
    o_ref[0] = jnp.sum(jnp.where(keep, sq, 0.0)) * _WEIGHT


@jax.jit
def kernel(pose):
    xt = pose.T
    out = pl.pallas_call(
        _tc_body,
        grid=(1,),
        in_specs=[pl.BlockSpec(memory_space=pl.ANY)],
        out_specs=pl.BlockSpec(memory_space=pltpu.SMEM),
        out_shape=jax.ShapeDtypeStruct((1,), jnp.float32),
        scratch_shapes=[
            pltpu.VMEM((8, _H), jnp.float32),
            pltpu.SemaphoreType.DMA,
        ],
    )(xt)
    return out[0]
